# Initial kernel scaffold; baseline (speedup 1.0000x reference)
import jax
import jax.numpy as jnp
from jax.experimental import pallas as pl


def kernel(x_seq, edge_index_seq, W0, lstm_Wi, lstm_Wh, lstm_b, W1, b1, W2, b2):
    raise NotImplementedError("write your pallas kernel here")


# speedup vs baseline: 199.8853x; 199.8853x over previous
import functools

import jax
import jax.numpy as jnp
from jax import lax
from jax.experimental import pallas as pl
from jax.experimental.pallas import tpu as pltpu
from jax.experimental.pallas import tpu_sc as plsc

N_NODES = 100000
E_EDGES = 1600000
NC, NS, L = 2, 16, 16
NW = NC * NS
NP = 100352
NODES_PER_TILE = NP // NS
E_PER_TILE = E_EDGES // NS
E_PER_WORKER = E_EDGES // NW
CHUNK = 25000

_MESH = plsc.VectorSubcoreMesh(
    core_axis_name="c", subcore_axis_name="s", num_cores=NC, num_subcores=NS)


@functools.partial(
    pl.kernel,
    out_type=(
        jax.ShapeDtypeStruct((NC, NP), jnp.float32),
        jax.ShapeDtypeStruct((NP,), jnp.float32),
    ),
    mesh=_MESH,
    scratch_types=[
        pltpu.VMEM((CHUNK,), jnp.int32),
        pltpu.VMEM((CHUNK,), jnp.int32),
        pltpu.VMEM((CHUNK,), jnp.float32),
        pltpu.VMEM((NODES_PER_TILE,), jnp.float32),
        pltpu.VMEM_SHARED((NP,), jnp.float32),
        pltpu.VMEM_SHARED((NP,), jnp.float32),
        pltpu.VMEM_SHARED((NP,), jnp.float32),
    ],
)
def _sc_edge_kernel(src_hbm, dst_hbm, w_hbm, dis_hbm,
                    src_v, dst_v, val_v, node_v, deg_sh, w_sh, dis_sh):
    c = lax.axis_index("c")
    s = lax.axis_index("s")
    node_base = s * NODES_PER_TILE

    def fill(ref, value, n):
        vals = jnp.full((L,), value, jnp.float32)

        def body(i, carry):
            ref[pl.ds(i * L, L)] = vals
            return carry

        lax.fori_loop(0, n // L, body, 0)

    fill(node_v, 0.0, NODES_PER_TILE)
    pltpu.sync_copy(node_v, deg_sh.at[pl.ds(node_base, NODES_PER_TILE)])
    pltpu.sync_copy(node_v, w_sh.at[pl.ds(node_base, NODES_PER_TILE)])
    plsc.subcore_barrier()

    fill(val_v, 1.0, CHUNK)

    def hbody(i, carry):
        base = s * E_PER_TILE + i * CHUNK
        pltpu.sync_copy(dst_hbm.at[pl.ds(base, CHUNK)], dst_v)
        pltpu.sync_copy(val_v, deg_sh.at[dst_v], add=True)
        return carry

    lax.fori_loop(0, E_PER_TILE // CHUNK, hbody, 0)
    plsc.subcore_barrier()

    pltpu.sync_copy(deg_sh.at[pl.ds(node_base, NODES_PER_TILE)], node_v)

    def dbody(i, carry):
        d = node_v[pl.ds(i * L, L)] + 1.0
        h = 0.5 * d
        bits = lax.bitcast_convert_type(d, jnp.int32)
        y = lax.bitcast_convert_type(
            0x5F3759DF - lax.shift_right_logical(bits, 1), jnp.float32)
        y = y * (1.5 - h * y * y)
        y = y * (1.5 - h * y * y)
        y = y * (1.5 - h * y * y)
        node_v[pl.ds(i * L, L)] = y
        return carry

    lax.fori_loop(0, NODES_PER_TILE // L, dbody, 0)
    pltpu.sync_copy(node_v, dis_sh.at[pl.ds(node_base, NODES_PER_TILE)])

    @pl.when(c == 0)
    def _():
        pltpu.sync_copy(node_v, dis_hbm.at[pl.ds(node_base, NODES_PER_TILE)])

    plsc.subcore_barrier()

    wid = s * NC + c

    def ebody(i, carry):
        base = wid * E_PER_WORKER + i * CHUNK
        pltpu.sync_copy(src_hbm.at[pl.ds(base, CHUNK)], src_v)
        pltpu.sync_copy(dst_hbm.at[pl.ds(base, CHUNK)], dst_v)
        pltpu.sync_copy(dis_sh.at[dst_v], val_v)
        pltpu.sync_copy(val_v, w_sh.at[src_v], add=True)
        return carry

    lax.fori_loop(0, E_PER_WORKER // CHUNK, ebody, 0)
    plsc.subcore_barrier()

    pltpu.sync_copy(w_sh.at[pl.ds(node_base, NODES_PER_TILE)], node_v)
    pltpu.sync_copy(node_v, w_hbm.at[c, pl.ds(node_base, NODES_PER_TILE)])


def _tc_tail_body(w2_ref, dis_ref, x3_ref, W0_ref, Wi_ref, Wh_ref, b_ref,
                  W1_ref, b1_ref, W2_ref, b2_ref, out_ref):
    dis = dis_ref[...]
    w = w2_ref[0] + w2_ref[1]
    coef = dis * w + dis * dis
    s0 = jnp.sum(coef * x3_ref[0])
    s1 = jnp.sum(coef * x3_ref[1])
    s2 = jnp.sum(coef * x3_ref[2])

    W = W0_ref[...]
    h = W
    cst = jnp.zeros_like(W)
    for _ in range(8):
        gates = W @ Wi_ref[...] + h @ Wh_ref[...] + b_ref[...][None, :]
        i_g, f_g, g_g, o_g = jnp.split(gates, 4, axis=-1)
        cst = jax.nn.sigmoid(f_g) * cst + jax.nn.sigmoid(i_g) * jnp.tanh(g_g)
        h = jax.nn.sigmoid(o_g) * jnp.tanh(cst)
        W = h

    g = (s0 * W[0] + s1 * W[1] + s2 * W[2]) * (1.0 / N_NODES)
    hid = jnp.maximum(g[None, :] @ W1_ref[...] + b1_ref[...][None, :], 0.0)
    out_ref[...] = hid @ W2_ref[...] + b2_ref[...][None, :]


_tc_tail = pl.pallas_call(
    _tc_tail_body,
    out_shape=jax.ShapeDtypeStruct((1, 2), jnp.float32),
)


def kernel(x_seq, edge_index_seq, W0, lstm_Wi, lstm_Wh, lstm_b, W1, b1, W2, b2):
    src = edge_index_seq[-1, 0].astype(jnp.int32)
    dst = edge_index_seq[-1, 1].astype(jnp.int32)
    x3 = jnp.pad(x_seq[-1].T, ((0, 0), (0, NP - N_NODES)))
    x3 = x3.reshape(3, NP // 128, 128)

    w2, dis = _sc_edge_kernel(src, dst)

    return _tc_tail(w2.reshape(NC, NP // 128, 128), dis.reshape(NP // 128, 128),
                    x3, W0, lstm_Wi, lstm_Wh, lstm_b, W1, b1, W2, b2)
